# baseline (speedup 1.0000x reference)
import dataclasses
import functools

import jax
import jax.numpy as jnp
from jax import lax
from jax.experimental import pallas as pl
from jax.experimental.pallas import tpu as pltpu
from jax.experimental.pallas import tpu_sc as plsc

_L = 16
_NC = 2
_NS = 16
_NW = _NC * _NS
_BP = 1024


def _pass1_body(img_v, edg_v, lo_v, val_v, acc_v):
    lane9 = lax.iota(jnp.int32, _L) * 9

    @pl.loop(0, _BP // _L)
    def _(g):
        base = g * (9 * _L)
        idx0 = lane9 + base
        zero = jnp.zeros((_L,), jnp.float32)
        sd = zero
        st = zero
        cd = zero
        ct = zero
        vals = zero
        for j in range(9):
            idx = idx0 + j
            ev = plsc.load_gather(edg_v, [idx])
            iv = plsc.load_gather(img_v, [idx])
            dm = ev > 0.5
            nz = ev != 0.0
            sd = sd + jnp.where(dm, iv, 0.0)
            st = st + jnp.where(nz, iv, 0.0)
            cd = cd + jnp.where(dm, 1.0, 0.0)
            ct = ct + jnp.where(nz, 1.0, 0.0)
            if j == 4:
                vals = iv
        sa = st - sd
        ca = ct - cd
        md = sd / jnp.maximum(cd, 1.0)
        ma = sa / jnp.maximum(ca, 1.0)
        known = (cd > 0.0) & (ca > 0.0)
        lo = jnp.where(known, jnp.where(md > ma, 0.0, 1.0), 2.0)
        lo_v[pl.ds(g * _L, _L)] = lo
        val_v[pl.ds(g * _L, _L)] = vals
        black = lo == 0.0
        one = jnp.ones((_L,), jnp.float32)
        plsc.addupdate(acc_v.at[pl.ds(0, _L)], jnp.where(black, vals, 0.0))
        plsc.addupdate(acc_v.at[pl.ds(_L, _L)], jnp.where(black, one, 0.0))
        plsc.addupdate(acc_v.at[pl.ds(2 * _L, _L)], jnp.where(black, 0.0, vals))
        plsc.addupdate(acc_v.at[pl.ds(3 * _L, _L)], jnp.where(black, 0.0, one))


def _sc_pass1(img, edg, n):
    mesh = plsc.VectorSubcoreMesh(core_axis_name="c", subcore_axis_name="s")
    cp = pltpu.CompilerParams()
    if "needs_layout_passes" in pltpu.CompilerParams.__dataclass_fields__:
        cp = dataclasses.replace(cp, needs_layout_passes=False)

    @functools.partial(
        pl.kernel,
        compiler_params=cp,
        out_type=(
            jax.ShapeDtypeStruct((n,), jnp.float32),
            jax.ShapeDtypeStruct((n,), jnp.float32),
            jax.ShapeDtypeStruct((_NW, 4 * _L), jnp.float32),
        ),
        mesh=mesh,
        scratch_types=[pltpu.VMEM((4 * _L,), jnp.float32)],
    )
    def k(img_hbm, edg_hbm, lo_hbm, val_hbm, part_hbm, acc_v):
        zero = jnp.zeros((_L,), jnp.float32)
        for q in range(4):
            acc_v[pl.ds(q * _L, _L)] = zero

        def body(img_v, edg_v, lo_v, val_v):
            _pass1_body(img_v, edg_v, lo_v, val_v, acc_v)

        pltpu.emit_pipeline(
            body,
            grid=(n // _BP,),
            in_specs=[
                pl.BlockSpec((_BP * 9,), lambda i: (i,)),
                pl.BlockSpec((_BP * 9,), lambda i: (i,)),
            ],
            out_specs=[
                pl.BlockSpec((_BP,), lambda i: (i,)),
                pl.BlockSpec((_BP,), lambda i: (i,)),
            ],
            core_axis_name=("c", "s"),
            dimension_semantics=(pltpu.PARALLEL,),
        )(img_hbm, edg_hbm, lo_hbm, val_hbm)

        wid = lax.axis_index("s") * _NC + lax.axis_index("c")
        pltpu.sync_copy(acc_v, part_hbm.at[wid])

    return k(img, edg)


def _pass2_tc(lo_ref, val_ref, part_ref, out_ref):
    p = part_ref[...]
    sB = jnp.sum(p[:, 0 * _L:1 * _L])
    cB = jnp.sum(p[:, 1 * _L:2 * _L])
    sW = jnp.sum(p[:, 2 * _L:3 * _L])
    cW = jnp.sum(p[:, 3 * _L:4 * _L])
    avgB = sB / jnp.maximum(cB, 1.0)
    avgW = sW / jnp.maximum(cW, 1.0)
    lo = lo_ref[...]
    v = val_ref[...]
    resolved = jnp.where(jnp.abs(v - avgB) < jnp.abs(v - avgW), 0.0, 1.0)
    out_ref[...] = jnp.where(lo != 2.0, lo, resolved)


def kernel(image, edges_prob, gt):
    n = image.shape[0]
    H = gt.shape[0] - 2
    W = gt.shape[1] - 2
    img = image.reshape(-1)
    edg = edges_prob.reshape(-1)
    lo, vals, parts = _sc_pass1(img, edg, n)
    out = pl.pallas_call(
        _pass2_tc,
        out_shape=jax.ShapeDtypeStruct((H, W), jnp.float32),
    )(lo.reshape(H, W), vals.reshape(H, W), parts)
    return out


# speedup vs baseline: 12.3612x; 12.3612x over previous
import dataclasses
import functools

import jax
import jax.numpy as jnp
from jax import lax
from jax.experimental import pallas as pl
from jax.experimental.pallas import tpu as pltpu
from jax.experimental.pallas import tpu_sc as plsc

_L = 16
_NC = 2
_NS = 16
_NW = _NC * _NS
_BP = 2048


def _pass1_body(data_v, lo_v, val_v, acc_v):

    @pl.loop(0, _BP // _L)
    def _(g):
        sl = pl.ds(g * _L, _L)
        zero = jnp.zeros((_L,), jnp.float32)
        sd = zero
        st = zero
        cd = zero
        ct = zero
        vals = zero
        for j in range(9):
            ev = data_v[pl.ds((9 + j) * _BP + g * _L, _L)]
            iv = data_v[pl.ds(j * _BP + g * _L, _L)]
            dm = ev > 0.5
            nz = ev != 0.0
            sd = sd + jnp.where(dm, iv, 0.0)
            st = st + jnp.where(nz, iv, 0.0)
            cd = cd + jnp.where(dm, 1.0, 0.0)
            ct = ct + jnp.where(nz, 1.0, 0.0)
            if j == 4:
                vals = iv
        sa = st - sd
        ca = ct - cd
        md = sd / jnp.maximum(cd, 1.0)
        ma = sa / jnp.maximum(ca, 1.0)
        known = (cd > 0.0) & (ca > 0.0)
        lo = jnp.where(known, jnp.where(md > ma, 0.0, 1.0), 2.0)
        lo_v[sl] = lo
        val_v[sl] = vals
        black = lo == 0.0
        one = jnp.ones((_L,), jnp.float32)
        plsc.addupdate(acc_v.at[0, pl.ds(0, _L)], jnp.where(black, vals, 0.0))
        plsc.addupdate(acc_v.at[0, pl.ds(_L, _L)], jnp.where(black, one, 0.0))
        plsc.addupdate(acc_v.at[0, pl.ds(2 * _L, _L)],
                       jnp.where(black, 0.0, vals))
        plsc.addupdate(acc_v.at[0, pl.ds(3 * _L, _L)],
                       jnp.where(black, 0.0, one))


def _sc_pass1(data, n):
    mesh = plsc.VectorSubcoreMesh(core_axis_name="c", subcore_axis_name="s")
    cp = pltpu.CompilerParams()
    if "needs_layout_passes" in pltpu.CompilerParams.__dataclass_fields__:
        cp = dataclasses.replace(cp, needs_layout_passes=False)

    @functools.partial(
        pl.kernel,
        compiler_params=cp,
        out_type=(
            jax.ShapeDtypeStruct((n,), jnp.float32),
            jax.ShapeDtypeStruct((n,), jnp.float32),
            jax.ShapeDtypeStruct((2 * _NW, 128), jnp.float32),
        ),
        mesh=mesh,
        scratch_types=[pltpu.VMEM((2, 128), jnp.float32)],
    )
    def k(data_hbm, lo_hbm, val_hbm, part_hbm, acc_v):
        zero = jnp.zeros((_L,), jnp.float32)
        for q in range(16):
            acc_v[q // 8, pl.ds((q % 8) * _L, _L)] = zero

        def body(data_v, lo_v, val_v):
            _pass1_body(data_v, lo_v, val_v, acc_v)

        blk = pl.BlockSpec((_BP,), lambda i: (i,))
        pltpu.emit_pipeline(
            body,
            grid=(n // _BP,),
            in_specs=[pl.BlockSpec((18 * _BP,), lambda i: (i,))],
            out_specs=[blk, blk],
            core_axis_name=("c", "s"),
            dimension_semantics=(pltpu.PARALLEL,),
        )(data_hbm, lo_hbm, val_hbm)

        wid = lax.axis_index("s") * _NC + lax.axis_index("c")
        pltpu.sync_copy(acc_v, part_hbm.at[pl.ds(2 * wid, 2)])

    return k(data)


def _pass2_tc(lo_ref, val_ref, part_ref, out_ref):
    p = part_ref[...]
    sB = jnp.sum(p[:, 0 * _L:1 * _L])
    cB = jnp.sum(p[:, 1 * _L:2 * _L])
    sW = jnp.sum(p[:, 2 * _L:3 * _L])
    cW = jnp.sum(p[:, 3 * _L:4 * _L])
    avgB = sB / jnp.maximum(cB, 1.0)
    avgW = sW / jnp.maximum(cW, 1.0)
    lo = lo_ref[...]
    v = val_ref[...]
    resolved = jnp.where(jnp.abs(v - avgB) < jnp.abs(v - avgW), 0.0, 1.0)
    corr = jnp.where(lo != 2.0, lo, resolved)
    out_ref[...] = corr.reshape(out_ref.shape)


def kernel(image, edges_prob, gt):
    n = image.shape[0]
    H = gt.shape[0] - 2
    W = gt.shape[1] - 2
    planes = [image[:, i, j] for i in range(3) for j in range(3)]
    planes += [edges_prob[:, i, j] for i in range(3) for j in range(3)]
    data = jnp.stack([p.reshape(n // _BP, _BP) for p in planes],
                     axis=1).reshape(-1)
    lo, vals, parts = _sc_pass1(data, n)

    rows = 64
    out = pl.pallas_call(
        _pass2_tc,
        grid=(H // rows,),
        in_specs=[
            pl.BlockSpec((rows * W,), lambda i: (i,)),
            pl.BlockSpec((rows * W,), lambda i: (i,)),
            pl.BlockSpec((2 * _NW, 128), lambda i: (0, 0)),
        ],
        out_specs=pl.BlockSpec((rows, W), lambda i: (i, 0)),
        out_shape=jax.ShapeDtypeStruct((H, W), jnp.float32),
    )(lo, vals, parts)
    return out


# speedup vs baseline: 17.0916x; 1.3827x over previous
import dataclasses
import functools

import jax
import jax.numpy as jnp
from jax import lax
from jax.experimental import pallas as pl
from jax.experimental.pallas import tpu as pltpu
from jax.experimental.pallas import tpu_sc as plsc

_L = 16
_NC = 2
_NS = 16
_NW = _NC * _NS
_BP = 2048


def _pass1_body(img_v, edg_v, lo_v, val_v, acc_v):

    @pl.loop(0, _BP // _L)
    def _(g):
        sl = pl.ds(g * _L, _L)
        zero = jnp.zeros((_L,), jnp.float32)
        sd = zero
        st = zero
        cd = zero
        ct = zero
        vals = zero
        for j in range(9):
            ev = edg_v[pl.ds(j * _BP + g * _L, _L)]
            iv = img_v[pl.ds(j * _BP + g * _L, _L)]
            dm = ev > 0.5
            nz = ev != 0.0
            sd = sd + jnp.where(dm, iv, 0.0)
            st = st + jnp.where(nz, iv, 0.0)
            cd = cd + jnp.where(dm, 1.0, 0.0)
            ct = ct + jnp.where(nz, 1.0, 0.0)
            if j == 4:
                vals = iv
        sa = st - sd
        ca = ct - cd
        md = sd / jnp.maximum(cd, 1.0)
        ma = sa / jnp.maximum(ca, 1.0)
        known = (cd > 0.0) & (ca > 0.0)
        lo = jnp.where(known, jnp.where(md > ma, 0.0, 1.0), 2.0)
        lo_v[sl] = lo
        val_v[sl] = vals
        black = lo == 0.0
        one = jnp.ones((_L,), jnp.float32)
        plsc.addupdate(acc_v.at[0, pl.ds(0, _L)], jnp.where(black, vals, 0.0))
        plsc.addupdate(acc_v.at[0, pl.ds(_L, _L)], jnp.where(black, one, 0.0))
        plsc.addupdate(acc_v.at[0, pl.ds(2 * _L, _L)],
                       jnp.where(black, 0.0, vals))
        plsc.addupdate(acc_v.at[0, pl.ds(3 * _L, _L)],
                       jnp.where(black, 0.0, one))


def _sc_pass1(img_pl, edg_pl, n):
    mesh = plsc.VectorSubcoreMesh(core_axis_name="c", subcore_axis_name="s")
    cp = pltpu.CompilerParams()
    if "needs_layout_passes" in pltpu.CompilerParams.__dataclass_fields__:
        cp = dataclasses.replace(cp, needs_layout_passes=False)

    @functools.partial(
        pl.kernel,
        compiler_params=cp,
        out_type=(
            jax.ShapeDtypeStruct((n,), jnp.float32),
            jax.ShapeDtypeStruct((n,), jnp.float32),
            jax.ShapeDtypeStruct((2 * _NW, 128), jnp.float32),
        ),
        mesh=mesh,
        scratch_types=[pltpu.VMEM((2, 128), jnp.float32)],
    )
    def k(img_hbm, edg_hbm, lo_hbm, val_hbm, part_hbm, acc_v):
        zero = jnp.zeros((_L,), jnp.float32)
        for q in range(16):
            acc_v[q // 8, pl.ds((q % 8) * _L, _L)] = zero

        def body(img_v, edg_v, lo_v, val_v):
            _pass1_body(img_v, edg_v, lo_v, val_v, acc_v)

        blk = pl.BlockSpec((_BP,), lambda i: (i,))
        pblk = pl.BlockSpec((9 * _BP,), lambda i: (i,))
        pltpu.emit_pipeline(
            body,
            grid=(n // _BP,),
            in_specs=[pblk, pblk],
            out_specs=[blk, blk],
            core_axis_name=("c", "s"),
            dimension_semantics=(pltpu.PARALLEL,),
        )(img_hbm, edg_hbm, lo_hbm, val_hbm)

        wid = lax.axis_index("s") * _NC + lax.axis_index("c")
        pltpu.sync_copy(acc_v, part_hbm.at[pl.ds(2 * wid, 2)])

    return k(img_pl, edg_pl)


def _pass2_tc(lo_ref, val_ref, part_ref, out_ref):
    p = part_ref[...]
    sB = jnp.sum(p[:, 0 * _L:1 * _L])
    cB = jnp.sum(p[:, 1 * _L:2 * _L])
    sW = jnp.sum(p[:, 2 * _L:3 * _L])
    cW = jnp.sum(p[:, 3 * _L:4 * _L])
    avgB = sB / jnp.maximum(cB, 1.0)
    avgW = sW / jnp.maximum(cW, 1.0)
    lo = lo_ref[...]
    v = val_ref[...]
    resolved = jnp.where(jnp.abs(v - avgB) < jnp.abs(v - avgW), 0.0, 1.0)
    corr = jnp.where(lo != 2.0, lo, resolved)
    out_ref[...] = corr.reshape(out_ref.shape)


def kernel(image, edges_prob, gt):
    n = image.shape[0]
    H = gt.shape[0] - 2
    W = gt.shape[1] - 2
    nb = n // _BP
    img_pl = image.reshape(nb, _BP, 9).transpose(0, 2, 1).reshape(-1)
    edg_pl = edges_prob.reshape(nb, _BP, 9).transpose(0, 2, 1).reshape(-1)
    lo, vals, parts = _sc_pass1(img_pl, edg_pl, n)

    rows = 64
    out = pl.pallas_call(
        _pass2_tc,
        grid=(H // rows,),
        in_specs=[
            pl.BlockSpec((rows * W,), lambda i: (i,)),
            pl.BlockSpec((rows * W,), lambda i: (i,)),
            pl.BlockSpec((2 * _NW, 128), lambda i: (0, 0)),
        ],
        out_specs=pl.BlockSpec((rows, W), lambda i: (i, 0)),
        out_shape=jax.ShapeDtypeStruct((H, W), jnp.float32),
    )(lo, vals, parts)
    return out
